# baseline (speedup 1.0000x reference)
import functools
import math

import jax
import jax.numpy as jnp
from jax import lax
from jax.experimental import pallas as pl
from jax.experimental.pallas import tpu as pltpu
from jax.experimental.pallas import tpu_sc as plsc

NC = 2
NS = 16
NW = NC * NS
LANES = 128
GROUPS = 16
CHUNK = GROUPS * LANES
ZROWS = 200
NBUF = 8
PRIME = 4


def _node_pad(n):
  blk = NS * ZROWS
  return ((n + 1 + blk - 1) // blk) * blk


def _hist_kernel(n_sp, rows_per_tile, n_chunks):
  mesh = plsc.VectorSubcoreMesh(core_axis_name="c", subcore_axis_name="s")
  zslice = n_sp // NS

  def body(dst_hbm, out_hbm, idx_v, ones_v, zb_v, hist_sp, hsem):
    c = lax.axis_index("c")
    s = lax.axis_index("s")
    wid = s * NC + c

    def fill_z(i, _):
      zb_v[pl.ds(i * 16, 16)] = jnp.zeros((16,), jnp.float32)
      return _
    lax.fori_loop(0, zslice // 16, fill_z, None)

    def fill_o(i, _):
      ones_v[pl.ds(i * 16, 16)] = jnp.ones((16,), jnp.float32)
      return _
    lax.fori_loop(0, LANES // 16, fill_o, None)

    pltpu.sync_copy(zb_v, hist_sp.at[pl.ds(s * zslice, zslice)])
    plsc.subcore_barrier()

    def chunk(i, _):
      row0 = (wid * n_chunks + i) * GROUPS
      pltpu.sync_copy(dst_hbm.at[pl.ds(row0, GROUPS)], idx_v)
      hs = []
      for j in range(GROUPS):
        hs.append(pltpu.async_copy(
            ones_v, hist_sp.at[idx_v.at[j]], hsem, add=True))
      for h in hs:
        h.wait()
      return _
    lax.fori_loop(0, n_chunks, chunk, None)

    plsc.subcore_barrier()
    pltpu.sync_copy(hist_sp.at[pl.ds(s * zslice, zslice)],
                    out_hbm.at[c, pl.ds(s * zslice, zslice)])

  return pl.kernel(
      body,
      out_type=jax.ShapeDtypeStruct((NC, n_sp), jnp.float32),
      mesh=mesh,
      compiler_params=pltpu.CompilerParams(use_tc_tiling_on_sc=False),
      scratch_types=[
          pltpu.VMEM((GROUPS, LANES), jnp.int32),
          pltpu.VMEM((LANES,), jnp.float32),
          pltpu.VMEM((zslice,), jnp.float32),
          pltpu.VMEM_SHARED((n_sp,), jnp.float32),
          pltpu.SemaphoreType.DMA,
      ],
  )


def _agg_kernel(n_sp, d, n_chunks):
  mesh = plsc.VectorSubcoreMesh(core_axis_name="c", subcore_axis_name="s")
  zslice = n_sp // NS

  def body(src_hbm, dst_hbm, table_hbm, out_hbm,
           src_v, dst_v, rows_v, zb_v, acc_sp, *sems):
    c = lax.axis_index("c")
    s = lax.axis_index("s")
    wid = s * NC + c
    gsem = sems[:NBUF]
    ssem = sems[NBUF:]

    def fill_z(i, _):
      zb_v[i, :] = jnp.zeros((16,), jnp.float32)
      return _
    lax.fori_loop(0, ZROWS, fill_z, None)
    for k in range(zslice // ZROWS):
      pltpu.sync_copy(zb_v, acc_sp.at[pl.ds(s * zslice + k * ZROWS, ZROWS)])
    plsc.subcore_barrier()

    def chunk(i, _):
      row0 = (wid * n_chunks + i) * GROUPS
      pltpu.sync_copy(src_hbm.at[pl.ds(row0, GROUPS)], src_v)
      pltpu.sync_copy(dst_hbm.at[pl.ds(row0, GROUPS)], dst_v)
      g = [None] * NBUF
      sc = [None] * NBUF
      pend = [False] * NBUF
      for j in range(PRIME):
        g[j % NBUF] = pltpu.async_copy(
            table_hbm.at[src_v.at[j]], rows_v.at[j % NBUF], gsem[j % NBUF])
      for j in range(GROUPS):
        b = j % NBUF
        g[b].wait()
        sc[b] = pltpu.async_copy(
            rows_v.at[b], acc_sp.at[dst_v.at[j]], ssem[b], add=True)
        pend[b] = True
        nj = j + PRIME
        if nj < GROUPS:
          nb = nj % NBUF
          if pend[nb]:
            sc[nb].wait()
            pend[nb] = False
          g[nb] = pltpu.async_copy(
              table_hbm.at[src_v.at[nj]], rows_v.at[nb], gsem[nb])
      for b in range(NBUF):
        if pend[b]:
          sc[b].wait()
      return _
    lax.fori_loop(0, n_chunks, chunk, None)

    plsc.subcore_barrier()
    pltpu.sync_copy(acc_sp.at[pl.ds(s * zslice, zslice)],
                    out_hbm.at[c, pl.ds(s * zslice, zslice)])

  return pl.kernel(
      body,
      out_type=jax.ShapeDtypeStruct((NC, n_sp, d), jnp.float32),
      mesh=mesh,
      compiler_params=pltpu.CompilerParams(use_tc_tiling_on_sc=False),
      scratch_types=[
          pltpu.VMEM((GROUPS, LANES), jnp.int32),
          pltpu.VMEM((GROUPS, LANES), jnp.int32),
          pltpu.VMEM((NBUF, LANES, d), jnp.float32),
          pltpu.VMEM((ZROWS, d), jnp.float32),
          pltpu.VMEM_SHARED((n_sp, d), jnp.float32),
      ] + [pltpu.SemaphoreType.DMA] * (2 * NBUF),
  )


_BR = 2000


def _stage_a_body(dp0, dp1, x, w1, dis_o, g1_o):
  deg = dp0[:] + dp1[:] + 1.0
  dis = lax.rsqrt(deg)
  dis_o[:] = dis
  g1_o[:] = jnp.dot(x[:], w1[:], preferred_element_type=jnp.float32) * dis


def _stage_b_body(a0, a1, g1, dis, b1, u_o):
  total = (a0[:] + a1[:] + g1[:]) * dis[:] + b1[:]
  u_o[:] = jnp.maximum(total, 0.0) * dis[:]


def _stage_c_body(a0, a1, u, dis, w2, b2, out_o):
  v = (a0[:] + a1[:] + u[:]) * dis[:]
  o = jnp.dot(v, w2[:], preferred_element_type=jnp.float32) + b2[:]
  m = jnp.max(o, axis=1, keepdims=True)
  lse = m + jnp.log(jnp.sum(jnp.exp(o - m), axis=1, keepdims=True))
  out_o[:] = o - lse


def _row_spec(d):
  return pl.BlockSpec((_BR, d), lambda i: (i, 0))


def _full_spec(r, d):
  return pl.BlockSpec((r, d), lambda i: (0, 0))


def kernel(x, edge_index, W1, b1, W2, b2):
  n, d_in = x.shape
  e = edge_index.shape[1]
  d_hid = W1.shape[1]
  d_out = W2.shape[1]
  n_sp = _node_pad(n)
  pad_row = n

  n_chunks = math.ceil(e / (NW * CHUNK))
  e_pad = NW * n_chunks * CHUNK
  src = edge_index[0]
  dst = edge_index[1]
  if e_pad > e:
    src = jnp.concatenate([src, jnp.zeros((e_pad - e,), src.dtype)])
    dst = jnp.concatenate([dst, jnp.full((e_pad - e,), pad_row, dst.dtype)])
  src2 = src.reshape(-1, LANES)
  dst2 = dst.reshape(-1, LANES)

  hist = _hist_kernel(n_sp, n_sp // NS, n_chunks)(dst2)
  dp0 = hist[0, :n].reshape(n, 1)
  dp1 = hist[1, :n].reshape(n, 1)

  grid = (n // _BR,)
  dis, g1 = pl.pallas_call(
      _stage_a_body,
      grid=grid,
      in_specs=[_row_spec(1), _row_spec(1), _row_spec(d_in),
                _full_spec(d_in, d_hid)],
      out_specs=[_row_spec(1), _row_spec(d_hid)],
      out_shape=[jax.ShapeDtypeStruct((n, 1), jnp.float32),
                 jax.ShapeDtypeStruct((n, d_hid), jnp.float32)],
  )(dp0, dp1, x, W1)

  agg = _agg_kernel(n_sp, d_hid, n_chunks)
  acc1 = agg(src2, dst2, g1)

  u = pl.pallas_call(
      _stage_b_body,
      grid=grid,
      in_specs=[_row_spec(d_hid), _row_spec(d_hid), _row_spec(d_hid),
                _row_spec(1), _full_spec(1, d_hid)],
      out_specs=_row_spec(d_hid),
      out_shape=jax.ShapeDtypeStruct((n, d_hid), jnp.float32),
  )(acc1[0, :n], acc1[1, :n], g1, dis, b1.reshape(1, d_hid))

  acc2 = agg(src2, dst2, u)

  out = pl.pallas_call(
      _stage_c_body,
      grid=grid,
      in_specs=[_row_spec(d_hid), _row_spec(d_hid), _row_spec(d_hid),
                _row_spec(1), _full_spec(d_hid, d_out),
                _full_spec(1, d_out)],
      out_specs=_row_spec(d_out),
      out_shape=jax.ShapeDtypeStruct((n, d_out), jnp.float32),
  )(acc2[0, :n], acc2[1, :n], u, dis, W2, b2.reshape(1, d_out))

  return out


# speedup vs baseline: 1.2761x; 1.2761x over previous
import jax
import jax.numpy as jnp
from jax import lax
from jax.experimental import pallas as pl
from jax.experimental.pallas import tpu as pltpu
from jax.experimental.pallas import tpu_sc as plsc

NC = 2
NS = 16
NW = NC * NS
LANES = 128
NBUF = 6
PRIME = 3
HGROUPS = 22
AGROUPS = 25


def _hist_kernel(n_h, n_groups):
  mesh = plsc.VectorSubcoreMesh(core_axis_name="c", subcore_axis_name="s")
  zslice = n_h // NS
  per_tile = n_groups // NW
  n_extra = n_groups - per_tile * NW
  n_chunks = per_tile // HGROUPS
  rem = per_tile - n_chunks * HGROUPS

  def body(dst_hbm, out_hbm, idx_v, ones_v, zb_v, hist_sp, hsem):
    c = lax.axis_index("c")
    s = lax.axis_index("s")
    wid = s * NC + c

    def fill_z(i, _):
      zb_v[pl.ds(i * 16, 16)] = jnp.zeros((16,), jnp.float32)
      return _
    lax.fori_loop(0, zslice // 16, fill_z, None)

    def fill_o(i, _):
      ones_v[pl.ds(i * 16, 16)] = jnp.ones((16,), jnp.float32)
      return _
    lax.fori_loop(0, LANES // 16, fill_o, None)

    pltpu.sync_copy(zb_v, hist_sp.at[pl.ds(s * zslice, zslice)])
    plsc.subcore_barrier()

    def do_groups(row0, k):
      pltpu.sync_copy(dst_hbm.at[pl.ds(row0, k)], idx_v.at[pl.ds(0, k)])
      hs = []
      for j in range(k):
        hs.append(pltpu.async_copy(
            ones_v, hist_sp.at[idx_v.at[j]], hsem, add=True))
      for h in hs:
        h.wait()

    def chunk(i, _):
      do_groups(wid * per_tile + i * HGROUPS, HGROUPS)
      return _
    lax.fori_loop(0, n_chunks, chunk, None)
    if rem:
      do_groups(wid * per_tile + n_chunks * HGROUPS, rem)
    if n_extra:
      @pl.when(wid < n_extra)
      def _():
        do_groups(NW * per_tile + wid, 1)

    plsc.subcore_barrier()
    pltpu.sync_copy(hist_sp.at[pl.ds(s * zslice, zslice)],
                    out_hbm.at[c, pl.ds(s * zslice, zslice)])

  return pl.kernel(
      body,
      out_type=jax.ShapeDtypeStruct((NC, n_h), jnp.float32),
      mesh=mesh,
      compiler_params=pltpu.CompilerParams(use_tc_tiling_on_sc=False),
      scratch_types=[
          pltpu.VMEM((HGROUPS, LANES), jnp.int32),
          pltpu.VMEM((LANES,), jnp.float32),
          pltpu.VMEM((zslice,), jnp.float32),
          pltpu.VMEM_SHARED((n_h,), jnp.float32),
          pltpu.SemaphoreType.DMA,
      ],
  )


def _agg_kernel(n, dh, n_groups):
  mesh = plsc.VectorSubcoreMesh(core_axis_name="c", subcore_axis_name="s")
  nslice = n // NS
  per_tile = n_groups // NS
  n_chunks = per_tile // AGROUPS
  rem = per_tile - n_chunks * AGROUPS

  def body(src_hbm, dst_hbm, table_hbm, zeros_hbm, out_hbm,
           src_v, dst_v, rows_v, table_sp, acc_sp, *sems):
    c = lax.axis_index("c")
    s = lax.axis_index("s")
    gsem = sems[:NBUF]
    ssem = sems[NBUF:]

    pltpu.sync_copy(zeros_hbm, acc_sp.at[pl.ds(s * nslice, nslice)])
    pltpu.sync_copy(table_hbm.at[c, pl.ds(s * nslice, nslice)],
                    table_sp.at[pl.ds(s * nslice, nslice)])
    plsc.subcore_barrier()

    def do_groups(row0, k):
      pltpu.sync_copy(src_hbm.at[pl.ds(row0, k)], src_v.at[pl.ds(0, k)])
      pltpu.sync_copy(dst_hbm.at[pl.ds(row0, k)], dst_v.at[pl.ds(0, k)])
      g = [None] * NBUF
      sc = [None] * NBUF
      pend = [False] * NBUF
      prime = min(PRIME, k)
      for j in range(prime):
        g[j % NBUF] = pltpu.async_copy(
            table_sp.at[src_v.at[j]], rows_v.at[j % NBUF], gsem[j % NBUF])
      for j in range(k):
        b = j % NBUF
        g[b].wait()
        sc[b] = pltpu.async_copy(
            rows_v.at[b], acc_sp.at[dst_v.at[j]], ssem[b], add=True)
        pend[b] = True
        nj = j + prime
        if nj < k:
          nb = nj % NBUF
          if pend[nb]:
            sc[nb].wait()
            pend[nb] = False
          g[nb] = pltpu.async_copy(
              table_sp.at[src_v.at[nj]], rows_v.at[nb], gsem[nb])
      for b in range(NBUF):
        if pend[b]:
          sc[b].wait()

    def chunk(i, _):
      do_groups(s * per_tile + i * AGROUPS, AGROUPS)
      return _
    lax.fori_loop(0, n_chunks, chunk, None)
    if rem:
      do_groups(s * per_tile + n_chunks * AGROUPS, rem)

    plsc.subcore_barrier()
    pltpu.sync_copy(acc_sp.at[pl.ds(s * nslice, nslice)],
                    out_hbm.at[c, pl.ds(s * nslice, nslice)])

  return pl.kernel(
      body,
      out_type=jax.ShapeDtypeStruct((NC, n, dh), jnp.float32),
      mesh=mesh,
      compiler_params=pltpu.CompilerParams(use_tc_tiling_on_sc=False),
      scratch_types=[
          pltpu.VMEM((AGROUPS, LANES), jnp.int32),
          pltpu.VMEM((AGROUPS, LANES), jnp.int32),
          pltpu.VMEM((NBUF, LANES, dh), jnp.float32),
          pltpu.VMEM_SHARED((n, dh), jnp.float32),
          pltpu.VMEM_SHARED((n, dh), jnp.float32),
      ] + [pltpu.SemaphoreType.DMA] * (2 * NBUF),
  )


_BR = 2000


def _stage_a_body(dp0, dp1, x, w1, dis_o, g_o):
  deg = dp0[:] + dp1[:] + 1.0
  dis = lax.rsqrt(deg)
  dis_o[:] = dis
  g = jnp.dot(x[:], w1[:], preferred_element_type=jnp.float32) * dis
  dh = g.shape[1] // 2
  g_o[0] = g[:, :dh]
  g_o[1] = g[:, dh:]


def _stage_b_body(acc, g, dis, b1, u_o):
  a16 = jnp.concatenate([acc[0] + g[0], acc[1] + g[1]], axis=1)
  u = jnp.maximum(a16 * dis[:] + b1[:], 0.0) * dis[:]
  dh = u.shape[1] // 2
  u_o[0] = u[:, :dh]
  u_o[1] = u[:, dh:]


def _stage_c_body(acc, u, dis, w2, b2, out_o):
  v = jnp.concatenate([acc[0] + u[0], acc[1] + u[1]], axis=1) * dis[:]
  o = jnp.dot(v, w2[:], preferred_element_type=jnp.float32) + b2[:]
  m = jnp.max(o, axis=1, keepdims=True)
  lse = m + jnp.log(jnp.sum(jnp.exp(o - m), axis=1, keepdims=True))
  out_o[:] = o - lse


def _row_spec(d):
  return pl.BlockSpec((_BR, d), lambda i: (i, 0))


def _split_spec(d):
  return pl.BlockSpec((NC, _BR, d), lambda i: (0, i, 0))


def _full_spec(r, d):
  return pl.BlockSpec((r, d), lambda i: (0, 0))


def kernel(x, edge_index, W1, b1, W2, b2):
  n, d_in = x.shape
  e = edge_index.shape[1]
  d_hid = W1.shape[1]
  d_out = W2.shape[1]
  dh = d_hid // 2
  n_h = ((n + 8 * NS - 1) // (8 * NS)) * (8 * NS)
  n_groups = e // LANES

  src2 = edge_index[0].reshape(n_groups, LANES)
  dst2 = edge_index[1].reshape(n_groups, LANES)

  hist = _hist_kernel(n_h, n_groups)(dst2)
  dp0 = hist[0, :n].reshape(n, 1)
  dp1 = hist[1, :n].reshape(n, 1)

  grid = (n // _BR,)
  dis, g1 = pl.pallas_call(
      _stage_a_body,
      grid=grid,
      in_specs=[_row_spec(1), _row_spec(1), _row_spec(d_in),
                _full_spec(d_in, d_hid)],
      out_specs=[_row_spec(1), _split_spec(dh)],
      out_shape=[jax.ShapeDtypeStruct((n, 1), jnp.float32),
                 jax.ShapeDtypeStruct((NC, n, dh), jnp.float32)],
  )(dp0, dp1, x, W1)

  agg = _agg_kernel(n, dh, n_groups)
  zrows = jnp.zeros((n // NS, dh), jnp.float32)
  acc1 = agg(src2, dst2, g1, zrows)

  u = pl.pallas_call(
      _stage_b_body,
      grid=grid,
      in_specs=[_split_spec(dh), _split_spec(dh), _row_spec(1),
                _full_spec(1, d_hid)],
      out_specs=_split_spec(dh),
      out_shape=jax.ShapeDtypeStruct((NC, n, dh), jnp.float32),
  )(acc1, g1, dis, b1.reshape(1, d_hid))

  acc2 = agg(src2, dst2, u, zrows)

  out = pl.pallas_call(
      _stage_c_body,
      grid=grid,
      in_specs=[_split_spec(dh), _split_spec(dh), _row_spec(1),
                _full_spec(d_hid, d_out), _full_spec(1, d_out)],
      out_specs=_row_spec(d_out),
      out_shape=jax.ShapeDtypeStruct((n, d_out), jnp.float32),
  )(acc2, u, dis, W2, b2.reshape(1, d_out))

  return out


# speedup vs baseline: 1.2820x; 1.0047x over previous
import jax
import jax.numpy as jnp
from jax import lax
from jax.experimental import pallas as pl
from jax.experimental.pallas import tpu as pltpu
from jax.experimental.pallas import tpu_sc as plsc

NC = 2
NS = 16
NW = NC * NS
LANES = 128
NBUF = 6
PRIME = 3
HGROUPS = 22
AGROUPS = 25


def _hist_kernel(n_h, n_groups):
  mesh = plsc.VectorSubcoreMesh(core_axis_name="c", subcore_axis_name="s")
  zslice = n_h // NS
  per_tile = n_groups // NW
  n_extra = n_groups - per_tile * NW
  n_chunks = per_tile // HGROUPS
  rem = per_tile - n_chunks * HGROUPS

  def body(edges_hbm, out_hbm, idx_v, ones_v, zb_v, hist_sp, hsem):
    c = lax.axis_index("c")
    s = lax.axis_index("s")
    wid = s * NC + c

    def fill_z(i, _):
      zb_v[pl.ds(i * 16, 16)] = jnp.zeros((16,), jnp.float32)
      return _
    lax.fori_loop(0, zslice // 16, fill_z, None)

    def fill_o(i, _):
      ones_v[pl.ds(i * 16, 16)] = jnp.ones((16,), jnp.float32)
      return _
    lax.fori_loop(0, LANES // 16, fill_o, None)

    pltpu.sync_copy(zb_v, hist_sp.at[pl.ds(s * zslice, zslice)])
    plsc.subcore_barrier()

    def do_groups(row0, k):
      pltpu.sync_copy(edges_hbm.at[1, pl.ds(row0, k)], idx_v.at[pl.ds(0, k)])
      hs = []
      for j in range(k):
        hs.append(pltpu.async_copy(
            ones_v, hist_sp.at[idx_v.at[j]], hsem, add=True))
      for h in hs:
        h.wait()

    def chunk(i, _):
      do_groups(wid * per_tile + i * HGROUPS, HGROUPS)
      return _
    lax.fori_loop(0, n_chunks, chunk, None)
    if rem:
      do_groups(wid * per_tile + n_chunks * HGROUPS, rem)
    if n_extra:
      @pl.when(wid < n_extra)
      def _():
        do_groups(NW * per_tile + wid, 1)

    plsc.subcore_barrier()
    pltpu.sync_copy(hist_sp.at[pl.ds(s * zslice, zslice)],
                    out_hbm.at[c, pl.ds(s * zslice, zslice)])

  return pl.kernel(
      body,
      out_type=jax.ShapeDtypeStruct((NC, n_h), jnp.float32),
      mesh=mesh,
      compiler_params=pltpu.CompilerParams(use_tc_tiling_on_sc=False),
      scratch_types=[
          pltpu.VMEM((HGROUPS, LANES), jnp.int32),
          pltpu.VMEM((LANES,), jnp.float32),
          pltpu.VMEM((zslice,), jnp.float32),
          pltpu.VMEM_SHARED((n_h,), jnp.float32),
          pltpu.SemaphoreType.DMA,
      ],
  )


def _agg_kernel(n, dh, n_groups):
  mesh = plsc.VectorSubcoreMesh(core_axis_name="c", subcore_axis_name="s")
  nslice = n // NS
  per_tile = n_groups // NS
  n_chunks = per_tile // AGROUPS
  rem = per_tile - n_chunks * AGROUPS

  def body(edges_hbm, table_hbm, zeros_hbm, out_hbm,
           src_v, dst_v, rows_v, table_sp, acc_sp, *sems):
    c = lax.axis_index("c")
    s = lax.axis_index("s")
    gsem = sems[:NBUF]
    ssem = sems[NBUF:]

    pltpu.sync_copy(zeros_hbm, acc_sp.at[pl.ds(s * nslice, nslice)])
    pltpu.sync_copy(table_hbm.at[c, pl.ds(s * nslice, nslice)],
                    table_sp.at[pl.ds(s * nslice, nslice)])
    plsc.subcore_barrier()

    def do_groups(row0, k):
      pltpu.sync_copy(edges_hbm.at[0, pl.ds(row0, k)], src_v.at[pl.ds(0, k)])
      pltpu.sync_copy(edges_hbm.at[1, pl.ds(row0, k)], dst_v.at[pl.ds(0, k)])
      g = [None] * NBUF
      sc = [None] * NBUF
      pend = [False] * NBUF
      prime = min(PRIME, k)
      for j in range(prime):
        g[j % NBUF] = pltpu.async_copy(
            table_sp.at[src_v.at[j]], rows_v.at[j % NBUF], gsem[j % NBUF])
      for j in range(k):
        b = j % NBUF
        g[b].wait()
        sc[b] = pltpu.async_copy(
            rows_v.at[b], acc_sp.at[dst_v.at[j]], ssem[b], add=True)
        pend[b] = True
        nj = j + prime
        if nj < k:
          nb = nj % NBUF
          if pend[nb]:
            sc[nb].wait()
            pend[nb] = False
          g[nb] = pltpu.async_copy(
              table_sp.at[src_v.at[nj]], rows_v.at[nb], gsem[nb])
      for b in range(NBUF):
        if pend[b]:
          sc[b].wait()

    def chunk(i, _):
      do_groups(s * per_tile + i * AGROUPS, AGROUPS)
      return _
    lax.fori_loop(0, n_chunks, chunk, None)
    if rem:
      do_groups(s * per_tile + n_chunks * AGROUPS, rem)

    plsc.subcore_barrier()
    pltpu.sync_copy(acc_sp.at[pl.ds(s * nslice, nslice)],
                    out_hbm.at[c, pl.ds(s * nslice, nslice)])

  return pl.kernel(
      body,
      out_type=jax.ShapeDtypeStruct((NC, n, dh), jnp.float32),
      mesh=mesh,
      compiler_params=pltpu.CompilerParams(use_tc_tiling_on_sc=False),
      scratch_types=[
          pltpu.VMEM((AGROUPS, LANES), jnp.int32),
          pltpu.VMEM((AGROUPS, LANES), jnp.int32),
          pltpu.VMEM((NBUF, LANES, dh), jnp.float32),
          pltpu.VMEM_SHARED((n, dh), jnp.float32),
          pltpu.VMEM_SHARED((n, dh), jnp.float32),
      ] + [pltpu.SemaphoreType.DMA] * (2 * NBUF),
  )


_BR = 5000


def _stage_a_body(hist, x, w1, dis_o, g_o):
  deg = hist[0] + hist[1] + 1.0
  dis = lax.rsqrt(deg)
  dis_o[:] = dis
  g = jnp.dot(x[:], w1[:], preferred_element_type=jnp.float32) * dis
  dh = g.shape[1] // 2
  g_o[0] = g[:, :dh]
  g_o[1] = g[:, dh:]


def _stage_b_body(acc, g, dis, b1, u_o):
  a16 = jnp.concatenate([acc[0] + g[0], acc[1] + g[1]], axis=1)
  u = jnp.maximum(a16 * dis[:] + b1[:], 0.0) * dis[:]
  dh = u.shape[1] // 2
  u_o[0] = u[:, :dh]
  u_o[1] = u[:, dh:]


def _stage_c_body(acc, u, dis, w2, b2, out_o):
  v = jnp.concatenate([acc[0] + u[0], acc[1] + u[1]], axis=1) * dis[:]
  o = jnp.dot(v, w2[:], preferred_element_type=jnp.float32) + b2[:]
  m = jnp.max(o, axis=1, keepdims=True)
  lse = m + jnp.log(jnp.sum(jnp.exp(o - m), axis=1, keepdims=True))
  out_o[:] = o - lse


def _row_spec(d):
  return pl.BlockSpec((_BR, d), lambda i: (i, 0))


def _split_spec(d):
  return pl.BlockSpec((NC, _BR, d), lambda i: (0, i, 0))


def _full_spec(r, d):
  return pl.BlockSpec((r, d), lambda i: (0, 0))


def kernel(x, edge_index, W1, b1, W2, b2):
  n, d_in = x.shape
  e = edge_index.shape[1]
  d_hid = W1.shape[1]
  d_out = W2.shape[1]
  dh = d_hid // 2
  n_h = ((n + 8 * NS - 1) // (8 * NS)) * (8 * NS)
  n_groups = e // LANES

  edges = edge_index.reshape(NC, n_groups, LANES)

  hist = _hist_kernel(n_h, n_groups)(edges)
  hist_rep = jnp.broadcast_to(hist[:, :, None], (NC, n_h, d_hid))

  grid = (n // _BR,)
  dis, g1 = pl.pallas_call(
      _stage_a_body,
      grid=grid,
      in_specs=[pl.BlockSpec((NC, _BR, d_hid), lambda i: (0, i, 0)),
                _row_spec(d_in), _full_spec(d_in, d_hid)],
      out_specs=[_row_spec(d_hid), _split_spec(dh)],
      out_shape=[jax.ShapeDtypeStruct((n, d_hid), jnp.float32),
                 jax.ShapeDtypeStruct((NC, n, dh), jnp.float32)],
  )(hist_rep, x, W1)

  agg = _agg_kernel(n, dh, n_groups)
  zrows = jnp.zeros((n // NS, dh), jnp.float32)
  acc1 = agg(edges, g1, zrows)

  u = pl.pallas_call(
      _stage_b_body,
      grid=grid,
      in_specs=[_split_spec(dh), _split_spec(dh), _row_spec(d_hid),
                _full_spec(1, d_hid)],
      out_specs=_split_spec(dh),
      out_shape=jax.ShapeDtypeStruct((NC, n, dh), jnp.float32),
  )(acc1, g1, dis, b1.reshape(1, d_hid))

  acc2 = agg(edges, u, zrows)

  out = pl.pallas_call(
      _stage_c_body,
      grid=grid,
      in_specs=[_split_spec(dh), _split_spec(dh), _row_spec(d_hid),
                _full_spec(d_hid, d_out), _full_spec(1, d_out)],
      out_specs=_row_spec(d_out),
      out_shape=jax.ShapeDtypeStruct((n, d_out), jnp.float32),
  )(acc2, u, dis, W2, b2.reshape(1, d_out))

  return out


# speedup vs baseline: 1.3257x; 1.0341x over previous
import jax
import jax.numpy as jnp
from jax import lax
from jax.experimental import pallas as pl
from jax.experimental.pallas import tpu as pltpu
from jax.experimental.pallas import tpu_sc as plsc

NC = 2
NS = 16
NW = NC * NS
LANES = 128
NBUF = 6
PRIME = 3
HGROUPS = 22
AGROUPS = 25


def _hist_kernel(n_h, n_groups):
  mesh = plsc.VectorSubcoreMesh(core_axis_name="c", subcore_axis_name="s")
  zslice = n_h // NS
  per_tile = n_groups // NW
  n_extra = n_groups - per_tile * NW
  n_chunks = per_tile // HGROUPS
  rem = per_tile - n_chunks * HGROUPS

  def body(edges_hbm, out_hbm, idx_v, ones_v, zb_v, hist_sp, hsem):
    c = lax.axis_index("c")
    s = lax.axis_index("s")
    wid = s * NC + c

    def fill_z(i, _):
      zb_v[pl.ds(i * 16, 16)] = jnp.zeros((16,), jnp.float32)
      return _
    lax.fori_loop(0, zslice // 16, fill_z, None)

    def fill_o(i, _):
      ones_v[pl.ds(i * 16, 16)] = jnp.ones((16,), jnp.float32)
      return _
    lax.fori_loop(0, LANES // 16, fill_o, None)

    pltpu.sync_copy(zb_v, hist_sp.at[pl.ds(s * zslice, zslice)])
    plsc.subcore_barrier()

    def do_groups(row0, k):
      pltpu.sync_copy(edges_hbm.at[1, pl.ds(row0, k)], idx_v.at[pl.ds(0, k)])
      hs = []
      for j in range(k):
        hs.append(pltpu.async_copy(
            ones_v, hist_sp.at[idx_v.at[j]], hsem, add=True))
      for h in hs:
        h.wait()

    def chunk(i, _):
      do_groups(wid * per_tile + i * HGROUPS, HGROUPS)
      return _
    lax.fori_loop(0, n_chunks, chunk, None)
    if rem:
      do_groups(wid * per_tile + n_chunks * HGROUPS, rem)
    if n_extra:
      @pl.when(wid < n_extra)
      def _():
        do_groups(NW * per_tile + wid, 1)

    plsc.subcore_barrier()
    pltpu.sync_copy(hist_sp.at[pl.ds(s * zslice, zslice)],
                    out_hbm.at[c, pl.ds(s * zslice, zslice)])

  return pl.kernel(
      body,
      out_type=jax.ShapeDtypeStruct((NC, n_h), jnp.float32),
      mesh=mesh,
      compiler_params=pltpu.CompilerParams(use_tc_tiling_on_sc=False),
      scratch_types=[
          pltpu.VMEM((HGROUPS, LANES), jnp.int32),
          pltpu.VMEM((LANES,), jnp.float32),
          pltpu.VMEM((zslice,), jnp.float32),
          pltpu.VMEM_SHARED((n_h,), jnp.float32),
          pltpu.SemaphoreType.DMA,
      ],
  )


def _agg_kernel(n, dh, n_groups):
  mesh = plsc.VectorSubcoreMesh(core_axis_name="c", subcore_axis_name="s")
  nslice = n // NS
  per_tile = n_groups // NS
  n_chunks = per_tile // AGROUPS
  rem = per_tile - n_chunks * AGROUPS

  def body(edges_hbm, table_hbm, zeros_hbm, out_hbm,
           src_v, dst_v, rows_v, table_sp, acc_sp, *sems):
    c = lax.axis_index("c")
    s = lax.axis_index("s")
    gsem = sems[:NBUF]
    ssem = sems[NBUF:]

    pltpu.sync_copy(zeros_hbm, acc_sp.at[pl.ds(s * nslice, nslice)])
    pltpu.sync_copy(table_hbm.at[c, pl.ds(s * nslice, nslice)],
                    table_sp.at[pl.ds(s * nslice, nslice)])
    plsc.subcore_barrier()

    def do_groups(row0, k):
      pltpu.sync_copy(edges_hbm.at[0, pl.ds(row0, k)], src_v.at[pl.ds(0, k)])
      pltpu.sync_copy(edges_hbm.at[1, pl.ds(row0, k)], dst_v.at[pl.ds(0, k)])
      g = [None] * NBUF
      sc = [None] * NBUF
      pend = [False] * NBUF
      prime = min(PRIME, k)
      for j in range(prime):
        g[j % NBUF] = pltpu.async_copy(
            table_sp.at[src_v.at[j]], rows_v.at[j % NBUF], gsem[j % NBUF])
      for j in range(k):
        b = j % NBUF
        g[b].wait()
        sc[b] = pltpu.async_copy(
            rows_v.at[b], acc_sp.at[dst_v.at[j]], ssem[b], add=True)
        pend[b] = True
        nj = j + prime
        if nj < k:
          nb = nj % NBUF
          if pend[nb]:
            sc[nb].wait()
            pend[nb] = False
          g[nb] = pltpu.async_copy(
              table_sp.at[src_v.at[nj]], rows_v.at[nb], gsem[nb])
      for b in range(NBUF):
        if pend[b]:
          sc[b].wait()

    def chunk(i, _):
      do_groups(s * per_tile + i * AGROUPS, AGROUPS)
      return _
    lax.fori_loop(0, n_chunks, chunk, None)
    if rem:
      do_groups(s * per_tile + n_chunks * AGROUPS, rem)

    plsc.subcore_barrier()
    pltpu.sync_copy(acc_sp.at[pl.ds(s * nslice, nslice)],
                    out_hbm.at[c, pl.ds(s * nslice, nslice)])

  return pl.kernel(
      body,
      out_type=jax.ShapeDtypeStruct((NC, n, dh), jnp.float32),
      mesh=mesh,
      compiler_params=pltpu.CompilerParams(use_tc_tiling_on_sc=False),
      scratch_types=[
          pltpu.VMEM((AGROUPS, LANES), jnp.int32),
          pltpu.VMEM((AGROUPS, LANES), jnp.int32),
          pltpu.VMEM((NBUF, LANES, dh), jnp.float32),
          pltpu.VMEM_SHARED((n, dh), jnp.float32),
          pltpu.VMEM_SHARED((n, dh), jnp.float32),
      ] + [pltpu.SemaphoreType.DMA] * (2 * NBUF),
  )


_BR = 6400


def _dis_col(hist):
  hpt = jnp.transpose(hist[:], (1, 0))
  deg = hpt[:, 0:1] + hpt[:, 1:2] + 1.0
  return lax.rsqrt(deg)


def _stage_a_body(hist, x, w1, g_o):
  dis = _dis_col(hist)
  g = jnp.dot(x[:], w1[:], preferred_element_type=jnp.float32) * dis
  dh = g.shape[1] // 2
  g_o[0] = g[:, :dh]
  g_o[1] = g[:, dh:]


def _stage_b_body(acc, g, hist, b1, u_o):
  dis = _dis_col(hist)
  a16 = jnp.concatenate([acc[0] + g[0], acc[1] + g[1]], axis=1)
  u = jnp.maximum(a16 * dis + b1[:], 0.0) * dis
  dh = u.shape[1] // 2
  u_o[0] = u[:, :dh]
  u_o[1] = u[:, dh:]


def _stage_c_body(acc, u, hist, w2, b2, out_o):
  dis = _dis_col(hist)
  v = jnp.concatenate([acc[0] + u[0], acc[1] + u[1]], axis=1) * dis
  o = jnp.dot(v, w2[:], preferred_element_type=jnp.float32) + b2[:]
  m = jnp.max(o, axis=1, keepdims=True)
  lse = m + jnp.log(jnp.sum(jnp.exp(o - m), axis=1, keepdims=True))
  out_o[:] = o - lse


def _row_spec(d):
  return pl.BlockSpec((_BR, d), lambda i: (i, 0))


def _split_spec(d):
  return pl.BlockSpec((NC, _BR, d), lambda i: (0, i, 0))


def _full_spec(r, d):
  return pl.BlockSpec((r, d), lambda i: (0, 0))


def kernel(x, edge_index, W1, b1, W2, b2):
  n0, d_in = x.shape
  e = edge_index.shape[1]
  d_hid = W1.shape[1]
  d_out = W2.shape[1]
  dh = d_hid // 2
  blk = NS * _BR
  n = ((n0 + blk - 1) // blk) * blk
  n_h = n
  n_groups = e // LANES

  x = jnp.concatenate([x, jnp.zeros((n - n0, d_in), x.dtype)]) if n > n0 else x
  edges = edge_index.reshape(NC, n_groups, LANES)

  hist = _hist_kernel(n_h, n_groups)(edges)
  hist_spec = pl.BlockSpec((NC, _BR), lambda i: (0, i))

  grid = (n // _BR,)
  g1 = pl.pallas_call(
      _stage_a_body,
      grid=grid,
      in_specs=[hist_spec, _row_spec(d_in), _full_spec(d_in, d_hid)],
      out_specs=_split_spec(dh),
      out_shape=jax.ShapeDtypeStruct((NC, n, dh), jnp.float32),
  )(hist, x, W1)

  agg = _agg_kernel(n, dh, n_groups)
  zrows = jnp.zeros((n // NS, dh), jnp.float32)
  acc1 = agg(edges, g1, zrows)

  u = pl.pallas_call(
      _stage_b_body,
      grid=grid,
      in_specs=[_split_spec(dh), _split_spec(dh), hist_spec,
                _full_spec(1, d_hid)],
      out_specs=_split_spec(dh),
      out_shape=jax.ShapeDtypeStruct((NC, n, dh), jnp.float32),
  )(acc1, g1, hist, b1.reshape(1, d_hid))

  acc2 = agg(edges, u, zrows)

  out = pl.pallas_call(
      _stage_c_body,
      grid=grid,
      in_specs=[_split_spec(dh), _split_spec(dh), hist_spec,
                _full_spec(d_hid, d_out), _full_spec(1, d_out)],
      out_specs=_row_spec(d_out),
      out_shape=jax.ShapeDtypeStruct((n, d_out), jnp.float32),
  )(acc2, u, hist, W2, b2.reshape(1, d_out))

  return out[:n0]
